# baseline (speedup 1.0000x reference)
import functools

import jax
import jax.numpy as jnp
from jax import lax
from jax.experimental import pallas as pl
from jax.experimental.pallas import tpu as pltpu
from jax.experimental.pallas import tpu_sc as plsc

N_NODES = 10000
N_PAD = 10240
N_EDGES = 320000
NCORES = 2
NSUB = 16
NW = NCORES * NSUB
BATCH = 64
GROUPS = 160
E_PAD = NW * GROUPS * BATCH
ROWS_PER_TILE = N_PAD // NSUB
PAD_SPREAD = 240


def _sc_prop(F):
    mesh = plsc.VectorSubcoreMesh(core_axis_name="c", subcore_axis_name="s")

    @functools.partial(
        pl.kernel,
        out_type=jax.ShapeDtypeStruct((NCORES, N_PAD, F), jnp.float32),
        mesh=mesh,
        scratch_types=[
            pltpu.VMEM_SHARED((N_PAD, F), jnp.float32),
            pltpu.VMEM((GROUPS // 2, 2 * BATCH), jnp.int32),
            pltpu.VMEM((GROUPS, BATCH), jnp.int32),
            pltpu.VMEM((BATCH, F), jnp.float32),
            pltpu.VMEM((BATCH, F), jnp.float32),
            pltpu.SemaphoreType.DMA,
            pltpu.SemaphoreType.DMA,
            pltpu.SemaphoreType.DMA,
            pltpu.SemaphoreType.DMA,
        ],
    )
    def prop(z_hbm, zeros_hbm, src_hbm, dst_hbm, out_hbm,
             acc, src_v, dst_v, rows0, rows1, g0s, g1s, s0s, s1s):
        c = lax.axis_index("c")
        s = lax.axis_index("s")
        w = c * NSUB + s
        pltpu.sync_copy(src_hbm.at[w], src_v)
        pltpu.sync_copy(dst_hbm.at[w], dst_v)
        r0 = s * ROWS_PER_TILE

        @pl.when(c == 0)
        def _():
            pltpu.sync_copy(z_hbm.at[pl.ds(r0, ROWS_PER_TILE)],
                            acc.at[pl.ds(r0, ROWS_PER_TILE)])

        @pl.when(c != 0)
        def _():
            pltpu.sync_copy(zeros_hbm.at[pl.ds(r0, ROWS_PER_TILE)],
                            acc.at[pl.ds(r0, ROWS_PER_TILE)])

        plsc.subcore_barrier()

        def gather(row, half, buf, sem):
            pltpu.async_copy(
                z_hbm.at[src_v.at[row, pl.ds(half * BATCH, BATCH)]], buf, sem)

        def wait_g(buf, sem):
            pltpu.make_async_copy(
                z_hbm.at[src_v.at[0, pl.ds(0, BATCH)]], buf, sem).wait()

        def scatter(g, buf, sem):
            pltpu.async_copy(buf, acc.at[dst_v.at[g]], sem, add=True)

        def wait_s(buf, sem):
            pltpu.make_async_copy(buf, acc.at[dst_v.at[0]], sem).wait()

        gather(0, 0, rows0, g0s)
        gather(0, 1, rows1, g1s)
        wait_g(rows0, g0s)
        scatter(0, rows0, s0s)
        wait_g(rows1, g1s)
        scatter(1, rows1, s1s)
        wait_s(rows0, s0s)
        gather(1, 0, rows0, g0s)
        def body(k, carry):
            wait_g(rows0, g0s)
            scatter(2 * k, rows0, s0s)
            wait_s(rows1, s1s)
            gather(k, 1, rows1, g1s)
            wait_g(rows1, g1s)
            scatter(2 * k + 1, rows1, s1s)
            wait_s(rows0, s0s)
            gather(k + 1, 0, rows0, g0s)
            return carry

        lax.fori_loop(1, GROUPS // 2 - 1, body, 0)
        k = GROUPS // 2 - 1
        wait_g(rows0, g0s)
        scatter(2 * k, rows0, s0s)
        wait_s(rows1, s1s)
        gather(k, 1, rows1, g1s)
        wait_g(rows1, g1s)
        scatter(2 * k + 1, rows1, s1s)
        wait_s(rows0, s0s)
        wait_s(rows1, s1s)
        plsc.subcore_barrier()
        pltpu.sync_copy(acc.at[pl.ds(r0, ROWS_PER_TILE)],
                        out_hbm.at[c, pl.ds(r0, ROWS_PER_TILE)])

    return prop


def _sc_deg():
    mesh = plsc.VectorSubcoreMesh(core_axis_name="c", subcore_axis_name="s")

    @functools.partial(
        pl.kernel,
        out_type=jax.ShapeDtypeStruct((NCORES, N_PAD, 128), jnp.float32),
        mesh=mesh,
        scratch_types=[
            pltpu.VMEM_SHARED((N_PAD, 128), jnp.float32),
            pltpu.VMEM((GROUPS, BATCH), jnp.int32),
            pltpu.VMEM((BATCH, 128), jnp.float32),
            pltpu.SemaphoreType.DMA,
        ],
    )
    def deg(dst_hbm, ones_hbm, zeros_hbm, out_hbm, acc, dst_v, ones_v, sem):
        c = lax.axis_index("c")
        s = lax.axis_index("s")
        w = c * NSUB + s
        pltpu.sync_copy(dst_hbm.at[w], dst_v)
        pltpu.sync_copy(ones_hbm, ones_v)
        r0 = s * ROWS_PER_TILE
        pltpu.sync_copy(zeros_hbm.at[pl.ds(r0, ROWS_PER_TILE)],
                        acc.at[pl.ds(r0, ROWS_PER_TILE)])
        plsc.subcore_barrier()

        def fire(g):
            pltpu.async_copy(ones_v, acc.at[dst_v.at[g]], sem, add=True)

        def drain():
            pltpu.make_async_copy(ones_v, acc.at[dst_v.at[0]], sem).wait()

        for g in range(4):
            fire(g)

        def body(g, carry):
            drain()
            fire(g)
            return carry

        lax.fori_loop(4, GROUPS, body, 0)
        for _ in range(4):
            drain()
        plsc.subcore_barrier()
        pltpu.sync_copy(acc.at[pl.ds(r0, ROWS_PER_TILE)],
                        out_hbm.at[c, pl.ds(r0, ROWS_PER_TILE)])

    return deg


_BM = 1024


def _dinv_from_deg(degp):
    def body(p_ref, o_ref):
        deg = p_ref[0, :, 0:1] + p_ref[1, :, 0:1] + 1.0
        safe = jnp.maximum(deg, 1e-12)
        o_ref[...] = jnp.where(deg > 0, lax.rsqrt(safe), 0.0)

    return pl.pallas_call(
        body,
        grid=(N_PAD // _BM,),
        in_specs=[pl.BlockSpec((2, _BM, 128), lambda i: (0, i, 0))],
        out_specs=pl.BlockSpec((_BM, 1), lambda i: (i, 0)),
        out_shape=jax.ShapeDtypeStruct((N_PAD, 1), jnp.float32),
    )(degp)


def _dense_in(h, W, b0, dinv, d0, apad):
    K = h.shape[1]
    dtot = W.shape[1]
    da = dtot - d0

    def body(h_ref, w_ref, b_ref, dv_ref, z0_ref, a_ref):
        prod = jnp.dot(h_ref[...], w_ref[...],
                       preferred_element_type=jnp.float32)
        z0_ref[...] = prod[:, :d0] + b_ref[...]
        av = prod[:, d0:] * dv_ref[...]
        if apad > da:
            av = jnp.concatenate(
                [av, jnp.zeros((av.shape[0], apad - da), jnp.float32)], axis=1)
        a_ref[...] = av

    return pl.pallas_call(
        body,
        grid=(N_PAD // _BM,),
        in_specs=[
            pl.BlockSpec((_BM, K), lambda i: (i, 0)),
            pl.BlockSpec((K, dtot), lambda i: (0, 0)),
            pl.BlockSpec((1, d0), lambda i: (0, 0)),
            pl.BlockSpec((_BM, 1), lambda i: (i, 0)),
        ],
        out_specs=[
            pl.BlockSpec((_BM, d0), lambda i: (i, 0)),
            pl.BlockSpec((_BM, apad), lambda i: (i, 0)),
        ],
        out_shape=[
            jax.ShapeDtypeStruct((N_PAD, d0), jnp.float32),
            jax.ShapeDtypeStruct((N_PAD, apad), jnp.float32),
        ],
    )(h, W, b0, dinv)


def _combine_mid(p, dinv, b1, dh):
    F = p.shape[2]
    da = F - dh

    def body(p_ref, dv_ref, b_ref, o1_ref, g2_ref):
        sm = p_ref[0] + p_ref[1]
        dv = dv_ref[...]
        o1_ref[...] = sm[:, :dh] * dv + b_ref[...]
        gv = sm[:, dh:] * (dv * dv)
        g2_ref[...] = jnp.concatenate(
            [gv, jnp.zeros((gv.shape[0], 128 - da), jnp.float32)], axis=1)

    return pl.pallas_call(
        body,
        grid=(N_PAD // _BM,),
        in_specs=[
            pl.BlockSpec((2, _BM, F), lambda i: (0, i, 0)),
            pl.BlockSpec((_BM, 1), lambda i: (i, 0)),
            pl.BlockSpec((1, dh), lambda i: (0, 0)),
        ],
        out_specs=[
            pl.BlockSpec((_BM, dh), lambda i: (i, 0)),
            pl.BlockSpec((_BM, 128), lambda i: (i, 0)),
        ],
        out_shape=[
            jax.ShapeDtypeStruct((N_PAD, dh), jnp.float32),
            jax.ShapeDtypeStruct((N_PAD, 128), jnp.float32),
        ],
    )(p, dinv, b1)


def _combine_out(z0, out1, q, dinv, b2, d2):
    d0 = z0.shape[1]
    d1 = out1.shape[1]
    Fq = q.shape[2]

    def body(z0_ref, o1_ref, q_ref, dv_ref, b_ref, h_ref):
        o2 = (q_ref[0, :, :d2] + q_ref[1, :, :d2]) * dv_ref[...] + b_ref[...]
        h_ref[...] = jnp.concatenate([z0_ref[...], o1_ref[...], o2], axis=1)

    return pl.pallas_call(
        body,
        grid=(N_PAD // _BM,),
        in_specs=[
            pl.BlockSpec((_BM, d0), lambda i: (i, 0)),
            pl.BlockSpec((_BM, d1), lambda i: (i, 0)),
            pl.BlockSpec((2, _BM, Fq), lambda i: (0, i, 0)),
            pl.BlockSpec((_BM, 1), lambda i: (i, 0)),
            pl.BlockSpec((1, d2), lambda i: (0, 0)),
        ],
        out_specs=pl.BlockSpec((_BM, d0 + d1 + d2), lambda i: (i, 0)),
        out_shape=jax.ShapeDtypeStruct((N_PAD, d0 + d1 + d2), jnp.float32),
    )(z0, out1, q, dinv, b2)


def _final_out(z0, q, dinv, b1, d1):
    d0 = z0.shape[1]
    Fq = q.shape[2]

    def body(z0_ref, q_ref, dv_ref, b_ref, h_ref):
        o1 = (q_ref[0, :, :d1] + q_ref[1, :, :d1]) * dv_ref[...] + b_ref[...]
        h_ref[...] = jnp.concatenate([z0_ref[...], o1], axis=1)

    return pl.pallas_call(
        body,
        grid=(N_PAD // _BM,),
        in_specs=[
            pl.BlockSpec((_BM, d0), lambda i: (i, 0)),
            pl.BlockSpec((2, _BM, Fq), lambda i: (0, i, 0)),
            pl.BlockSpec((_BM, 1), lambda i: (i, 0)),
            pl.BlockSpec((1, d1), lambda i: (0, 0)),
        ],
        out_specs=pl.BlockSpec((_BM, d0 + d1), lambda i: (i, 0)),
        out_shape=jax.ShapeDtypeStruct((N_PAD, d0 + d1), jnp.float32),
    )(z0, q, dinv, b1)


def kernel(x, edge_index, conv1_W, conv1_b, block_W, block_b, conv3_W, conv3_b):
    f32 = jnp.float32

    xp = jnp.pad(x, ((0, N_PAD - N_NODES), (0, 0)))
    npad = E_PAD - N_EDGES
    pad_ids = (jnp.arange(npad, dtype=jnp.int32) % PAD_SPREAD) + N_NODES
    srcp = jnp.concatenate([edge_index[0], pad_ids]).reshape(
        NW, GROUPS // 2, 2 * BATCH)
    dstp = jnp.concatenate([edge_index[1], pad_ids]).reshape(NW, GROUPS, BATCH)

    ones128 = jnp.ones((BATCH, 128), f32)
    zeros128 = jnp.zeros((N_PAD, 128), f32)

    degp = _sc_deg()(dstp, ones128, zeros128)
    dinv = _dinv_from_deg(degp)

    prop128 = _sc_prop(128)

    def mixhop3(h, Wcat, b0, b1, b2):
        z0, a = _dense_in(h, Wcat, b0, dinv, 64, 128)
        p = prop128(a, zeros128, srcp, dstp)
        out1, g2 = _combine_mid(p, dinv, b1, 64)
        q = prop128(g2, zeros128, srcp, dstp)
        return _combine_out(z0, out1, q, dinv, b2, 64)

    W1 = jnp.concatenate([conv1_W[0], conv1_W[1], conv1_W[2]], axis=1)
    h = mixhop3(xp, W1, conv1_b[0][None], conv1_b[1][None], conv1_b[2][None])

    for i in range(2):
        Wm = jnp.concatenate([block_W[i, 0], block_W[i, 1], block_W[i, 2]],
                             axis=1)
        h = mixhop3(h, Wm, block_b[i, 0][None], block_b[i, 1][None],
                    block_b[i, 2][None])

    W3 = jnp.concatenate([conv3_W[0], conv3_W[1]], axis=1)
    z0, a1 = _dense_in(h, W3, conv3_b[0][None], dinv, 32, 128)
    q = prop128(a1, zeros128, srcp, dstp)
    out = _final_out(z0, q, dinv, conv3_b[1][None], 32)
    return out[:N_NODES]


# speedup vs baseline: 1.1436x; 1.1436x over previous
import functools

import jax
import jax.numpy as jnp
from jax import lax
from jax.experimental import pallas as pl
from jax.experimental.pallas import tpu as pltpu
from jax.experimental.pallas import tpu_sc as plsc

N_NODES = 10000
N_PAD = 10240
N_EDGES = 320000
NCORES = 2
NSUB = 16
NW = NCORES * NSUB
BATCH = 64
GROUPS = 160
E_PAD = NW * GROUPS * BATCH
ROWS_PER_TILE = N_PAD // NSUB
PAD_SPREAD = 240


def _sc_prop(F, tc_tiling=True):
    mesh = plsc.VectorSubcoreMesh(core_axis_name="c", subcore_axis_name="s")

    @functools.partial(
        pl.kernel,
        out_type=jax.ShapeDtypeStruct((NCORES, N_PAD, F), jnp.float32),
        mesh=mesh,
        compiler_params=pltpu.CompilerParams(use_tc_tiling_on_sc=tc_tiling),
        scratch_types=[
            pltpu.VMEM_SHARED((N_PAD, F), jnp.float32),
            pltpu.VMEM((GROUPS // 2, 2 * BATCH), jnp.int32),
            pltpu.VMEM((GROUPS, BATCH), jnp.int32),
            pltpu.VMEM((BATCH, F), jnp.float32),
            pltpu.VMEM((BATCH, F), jnp.float32),
            pltpu.SemaphoreType.DMA,
            pltpu.SemaphoreType.DMA,
            pltpu.SemaphoreType.DMA,
            pltpu.SemaphoreType.DMA,
        ],
    )
    def prop(z_hbm, zeros_hbm, src_hbm, dst_hbm, out_hbm,
             acc, src_v, dst_v, rows0, rows1, g0s, g1s, s0s, s1s):
        c = lax.axis_index("c")
        s = lax.axis_index("s")
        w = c * NSUB + s
        pltpu.sync_copy(src_hbm.at[w], src_v)
        pltpu.sync_copy(dst_hbm.at[w], dst_v)
        r0 = s * ROWS_PER_TILE

        @pl.when(c == 0)
        def _():
            pltpu.sync_copy(z_hbm.at[pl.ds(r0, ROWS_PER_TILE)],
                            acc.at[pl.ds(r0, ROWS_PER_TILE)])

        @pl.when(c != 0)
        def _():
            pltpu.sync_copy(zeros_hbm.at[pl.ds(r0, ROWS_PER_TILE)],
                            acc.at[pl.ds(r0, ROWS_PER_TILE)])

        plsc.subcore_barrier()

        def gather(row, half, buf, sem):
            pltpu.async_copy(
                z_hbm.at[src_v.at[row, pl.ds(half * BATCH, BATCH)]], buf, sem)

        def wait_g(buf, sem):
            pltpu.make_async_copy(
                z_hbm.at[src_v.at[0, pl.ds(0, BATCH)]], buf, sem).wait()

        def scatter(g, buf, sem):
            pltpu.async_copy(buf, acc.at[dst_v.at[g]], sem, add=True)

        def wait_s(buf, sem):
            pltpu.make_async_copy(buf, acc.at[dst_v.at[0]], sem).wait()

        gather(0, 0, rows0, g0s)
        gather(0, 1, rows1, g1s)
        wait_g(rows0, g0s)
        scatter(0, rows0, s0s)
        wait_g(rows1, g1s)
        scatter(1, rows1, s1s)
        wait_s(rows0, s0s)
        gather(1, 0, rows0, g0s)
        def body(k, carry):
            wait_g(rows0, g0s)
            scatter(2 * k, rows0, s0s)
            wait_s(rows1, s1s)
            gather(k, 1, rows1, g1s)
            wait_g(rows1, g1s)
            scatter(2 * k + 1, rows1, s1s)
            wait_s(rows0, s0s)
            gather(k + 1, 0, rows0, g0s)
            return carry

        lax.fori_loop(1, GROUPS // 2 - 1, body, 0)
        k = GROUPS // 2 - 1
        wait_g(rows0, g0s)
        scatter(2 * k, rows0, s0s)
        wait_s(rows1, s1s)
        gather(k, 1, rows1, g1s)
        wait_g(rows1, g1s)
        scatter(2 * k + 1, rows1, s1s)
        wait_s(rows0, s0s)
        wait_s(rows1, s1s)
        plsc.subcore_barrier()
        pltpu.sync_copy(acc.at[pl.ds(r0, ROWS_PER_TILE)],
                        out_hbm.at[c, pl.ds(r0, ROWS_PER_TILE)])

    return prop


def _sc_deg():
    mesh = plsc.VectorSubcoreMesh(core_axis_name="c", subcore_axis_name="s")

    @functools.partial(
        pl.kernel,
        out_type=jax.ShapeDtypeStruct((NCORES, N_PAD, 16), jnp.float32),
        mesh=mesh,
        compiler_params=pltpu.CompilerParams(use_tc_tiling_on_sc=False),
        scratch_types=[
            pltpu.VMEM_SHARED((N_PAD, 16), jnp.float32),
            pltpu.VMEM((GROUPS, BATCH), jnp.int32),
            pltpu.VMEM((BATCH, 16), jnp.float32),
            pltpu.SemaphoreType.DMA,
        ],
    )
    def deg(dst_hbm, ones_hbm, zeros_hbm, out_hbm, acc, dst_v, ones_v, sem):
        c = lax.axis_index("c")
        s = lax.axis_index("s")
        w = c * NSUB + s
        pltpu.sync_copy(dst_hbm.at[w], dst_v)
        pltpu.sync_copy(ones_hbm, ones_v)
        r0 = s * ROWS_PER_TILE
        pltpu.sync_copy(zeros_hbm.at[pl.ds(r0, ROWS_PER_TILE)],
                        acc.at[pl.ds(r0, ROWS_PER_TILE)])
        plsc.subcore_barrier()

        def fire(g):
            pltpu.async_copy(ones_v, acc.at[dst_v.at[g]], sem, add=True)

        def drain():
            pltpu.make_async_copy(ones_v, acc.at[dst_v.at[0]], sem).wait()

        for g in range(4):
            fire(g)

        def body(g, carry):
            drain()
            fire(g)
            return carry

        lax.fori_loop(4, GROUPS, body, 0)
        for _ in range(4):
            drain()
        plsc.subcore_barrier()
        pltpu.sync_copy(acc.at[pl.ds(r0, ROWS_PER_TILE)],
                        out_hbm.at[c, pl.ds(r0, ROWS_PER_TILE)])

    return deg


_BM = 1024


def _dinv_from_deg(degp):
    def body(p_ref, o_ref):
        deg = p_ref[0, :, 0:1] + p_ref[1, :, 0:1] + 1.0
        safe = jnp.maximum(deg, 1e-12)
        o_ref[...] = jnp.where(deg > 0, lax.rsqrt(safe), 0.0)

    return pl.pallas_call(
        body,
        grid=(N_PAD // _BM,),
        in_specs=[pl.BlockSpec((2, _BM, 16), lambda i: (0, i, 0))],
        out_specs=pl.BlockSpec((_BM, 1), lambda i: (i, 0)),
        out_shape=jax.ShapeDtypeStruct((N_PAD, 1), jnp.float32),
    )(degp)


def _dense_in(h, W, b0, dinv, d0, apad):
    K = h.shape[1]
    dtot = W.shape[1]
    da = dtot - d0

    def body(h_ref, w_ref, b_ref, dv_ref, z0_ref, a_ref):
        prod = jnp.dot(h_ref[...], w_ref[...],
                       preferred_element_type=jnp.float32)
        z0_ref[...] = prod[:, :d0] + b_ref[...]
        av = prod[:, d0:] * dv_ref[...]
        if apad > da:
            av = jnp.concatenate(
                [av, jnp.zeros((av.shape[0], apad - da), jnp.float32)], axis=1)
        a_ref[...] = av

    return pl.pallas_call(
        body,
        grid=(N_PAD // _BM,),
        in_specs=[
            pl.BlockSpec((_BM, K), lambda i: (i, 0)),
            pl.BlockSpec((K, dtot), lambda i: (0, 0)),
            pl.BlockSpec((1, d0), lambda i: (0, 0)),
            pl.BlockSpec((_BM, 1), lambda i: (i, 0)),
        ],
        out_specs=[
            pl.BlockSpec((_BM, d0), lambda i: (i, 0)),
            pl.BlockSpec((_BM, apad), lambda i: (i, 0)),
        ],
        out_shape=[
            jax.ShapeDtypeStruct((N_PAD, d0), jnp.float32),
            jax.ShapeDtypeStruct((N_PAD, apad), jnp.float32),
        ],
    )(h, W, b0, dinv)


def _combine_mid(p, dinv, b1, dh):
    F = p.shape[2]
    da = F - dh

    def body(p_ref, dv_ref, b_ref, o1_ref, g2_ref):
        sm = p_ref[0] + p_ref[1]
        dv = dv_ref[...]
        o1_ref[...] = sm[:, :dh] * dv + b_ref[...]
        g2_ref[...] = sm[:, dh:] * (dv * dv)

    return pl.pallas_call(
        body,
        grid=(N_PAD // _BM,),
        in_specs=[
            pl.BlockSpec((2, _BM, F), lambda i: (0, i, 0)),
            pl.BlockSpec((_BM, 1), lambda i: (i, 0)),
            pl.BlockSpec((1, dh), lambda i: (0, 0)),
        ],
        out_specs=[
            pl.BlockSpec((_BM, dh), lambda i: (i, 0)),
            pl.BlockSpec((_BM, da), lambda i: (i, 0)),
        ],
        out_shape=[
            jax.ShapeDtypeStruct((N_PAD, dh), jnp.float32),
            jax.ShapeDtypeStruct((N_PAD, da), jnp.float32),
        ],
    )(p, dinv, b1)


def _combine_out(z0, out1, q, dinv, b2, d2):
    d0 = z0.shape[1]
    d1 = out1.shape[1]
    Fq = q.shape[2]

    def body(z0_ref, o1_ref, q_ref, dv_ref, b_ref, h_ref):
        o2 = (q_ref[0, :, :d2] + q_ref[1, :, :d2]) * dv_ref[...] + b_ref[...]
        h_ref[...] = jnp.concatenate([z0_ref[...], o1_ref[...], o2], axis=1)

    return pl.pallas_call(
        body,
        grid=(N_PAD // _BM,),
        in_specs=[
            pl.BlockSpec((_BM, d0), lambda i: (i, 0)),
            pl.BlockSpec((_BM, d1), lambda i: (i, 0)),
            pl.BlockSpec((2, _BM, Fq), lambda i: (0, i, 0)),
            pl.BlockSpec((_BM, 1), lambda i: (i, 0)),
            pl.BlockSpec((1, d2), lambda i: (0, 0)),
        ],
        out_specs=pl.BlockSpec((_BM, d0 + d1 + d2), lambda i: (i, 0)),
        out_shape=jax.ShapeDtypeStruct((N_PAD, d0 + d1 + d2), jnp.float32),
    )(z0, out1, q, dinv, b2)


def _final_out(z0, q, dinv, b1, d1):
    d0 = z0.shape[1]
    Fq = q.shape[2]

    def body(z0_ref, q_ref, dv_ref, b_ref, h_ref):
        o1 = (q_ref[0, :, :d1] + q_ref[1, :, :d1]) * dv_ref[...] + b_ref[...]
        h_ref[...] = jnp.concatenate([z0_ref[...], o1], axis=1)

    return pl.pallas_call(
        body,
        grid=(N_PAD // _BM,),
        in_specs=[
            pl.BlockSpec((_BM, d0), lambda i: (i, 0)),
            pl.BlockSpec((2, _BM, Fq), lambda i: (0, i, 0)),
            pl.BlockSpec((_BM, 1), lambda i: (i, 0)),
            pl.BlockSpec((1, d1), lambda i: (0, 0)),
        ],
        out_specs=pl.BlockSpec((_BM, d0 + d1), lambda i: (i, 0)),
        out_shape=jax.ShapeDtypeStruct((N_PAD, d0 + d1), jnp.float32),
    )(z0, q, dinv, b1)


def kernel(x, edge_index, conv1_W, conv1_b, block_W, block_b, conv3_W, conv3_b):
    f32 = jnp.float32

    xp = jnp.pad(x, ((0, N_PAD - N_NODES), (0, 0)))
    npad = E_PAD - N_EDGES
    pad_ids = (jnp.arange(npad, dtype=jnp.int32) % PAD_SPREAD) + N_NODES
    srcp = jnp.concatenate([edge_index[0], pad_ids]).reshape(
        NW, GROUPS // 2, 2 * BATCH)
    dstp = jnp.concatenate([edge_index[1], pad_ids]).reshape(NW, GROUPS, BATCH)

    ones16 = jnp.ones((BATCH, 16), f32)
    zeros16 = jnp.zeros((N_PAD, 16), f32)
    zeros128 = jnp.zeros((N_PAD, 128), f32)
    zeros64 = jnp.zeros((N_PAD, 64), f32)
    zeros32 = jnp.zeros((N_PAD, 32), f32)

    degp = _sc_deg()(dstp, ones16, zeros16)
    dinv = _dinv_from_deg(degp)

    prop128 = _sc_prop(128)
    prop64 = _sc_prop(64, tc_tiling=False)
    prop32 = _sc_prop(32, tc_tiling=False)

    def mixhop3(h, Wcat, b0, b1, b2):
        z0, a = _dense_in(h, Wcat, b0, dinv, 64, 128)
        p = prop128(a, zeros128, srcp, dstp)
        out1, g2 = _combine_mid(p, dinv, b1, 64)
        q = prop64(g2, zeros64, srcp, dstp)
        return _combine_out(z0, out1, q, dinv, b2, 64)

    W1 = jnp.concatenate([conv1_W[0], conv1_W[1], conv1_W[2]], axis=1)
    h = mixhop3(xp, W1, conv1_b[0][None], conv1_b[1][None], conv1_b[2][None])

    for i in range(2):
        Wm = jnp.concatenate([block_W[i, 0], block_W[i, 1], block_W[i, 2]],
                             axis=1)
        h = mixhop3(h, Wm, block_b[i, 0][None], block_b[i, 1][None],
                    block_b[i, 2][None])

    W3 = jnp.concatenate([conv3_W[0], conv3_W[1]], axis=1)
    z0, a1 = _dense_in(h, W3, conv3_b[0][None], dinv, 32, 32)
    q = prop32(a1, zeros32, srcp, dstp)
    out = _final_out(z0, q, dinv, conv3_b[1][None], 32)
    return out[:N_NODES]


# speedup vs baseline: 1.3768x; 1.2040x over previous
import functools

import jax
import jax.numpy as jnp
from jax import lax
from jax.experimental import pallas as pl
from jax.experimental.pallas import tpu as pltpu
from jax.experimental.pallas import tpu_sc as plsc

N_NODES = 10000
N_PAD = 10112
N_EDGES = 320000
NCORES = 2
NSUB = 16
BATCH = 64
GROUPS_P = 320
GROUPS_D = 160
E_PAD = NSUB * GROUPS_P * BATCH
RPT = N_PAD // NSUB
PAD_SPREAD = 112


def _sc_prop(HALF):
    mesh = plsc.VectorSubcoreMesh(core_axis_name="c", subcore_axis_name="s")

    @functools.partial(
        pl.kernel,
        out_type=jax.ShapeDtypeStruct((NCORES, N_PAD, HALF), jnp.float32),
        mesh=mesh,
        compiler_params=pltpu.CompilerParams(use_tc_tiling_on_sc=False),
        scratch_types=[
            pltpu.VMEM_SHARED((N_PAD, HALF), jnp.float32),
            pltpu.VMEM_SHARED((N_PAD, HALF), jnp.float32),
            pltpu.VMEM((GROUPS_P // 2, 2 * BATCH), jnp.int32),
            pltpu.VMEM((GROUPS_P, BATCH), jnp.int32),
            pltpu.VMEM((BATCH, HALF), jnp.float32),
            pltpu.VMEM((BATCH, HALF), jnp.float32),
            pltpu.SemaphoreType.DMA,
            pltpu.SemaphoreType.DMA,
            pltpu.SemaphoreType.DMA,
            pltpu.SemaphoreType.DMA,
        ],
    )
    def prop(zl_hbm, zr_hbm, src_hbm, dst_hbm, out_hbm,
             zsp, acc, src_v, dst_v, rows0, rows1, g0s, g1s, s0s, s1s):
        c = lax.axis_index("c")
        s = lax.axis_index("s")
        pltpu.sync_copy(src_hbm.at[s], src_v)
        pltpu.sync_copy(dst_hbm.at[s], dst_v)
        r0 = s * RPT

        @pl.when(c == 0)
        def _():
            pltpu.sync_copy(zl_hbm.at[pl.ds(r0, RPT)], zsp.at[pl.ds(r0, RPT)])
            pltpu.sync_copy(zl_hbm.at[pl.ds(r0, RPT)], acc.at[pl.ds(r0, RPT)])

        @pl.when(c != 0)
        def _():
            pltpu.sync_copy(zr_hbm.at[pl.ds(r0, RPT)], zsp.at[pl.ds(r0, RPT)])
            pltpu.sync_copy(zr_hbm.at[pl.ds(r0, RPT)], acc.at[pl.ds(r0, RPT)])

        plsc.subcore_barrier()

        def gather(row, half, buf, sem):
            pltpu.async_copy(
                zsp.at[src_v.at[row, pl.ds(half * BATCH, BATCH)]], buf, sem)

        def wait_g(buf, sem):
            pltpu.make_async_copy(
                zsp.at[src_v.at[0, pl.ds(0, BATCH)]], buf, sem).wait()

        def scatter(g, buf, sem):
            pltpu.async_copy(buf, acc.at[dst_v.at[g]], sem, add=True)

        def wait_s(buf, sem):
            pltpu.make_async_copy(buf, acc.at[dst_v.at[0]], sem).wait()

        gather(0, 0, rows0, g0s)
        gather(0, 1, rows1, g1s)
        wait_g(rows0, g0s)
        scatter(0, rows0, s0s)
        wait_g(rows1, g1s)
        scatter(1, rows1, s1s)
        wait_s(rows0, s0s)
        gather(1, 0, rows0, g0s)

        def body(k, carry):
            wait_g(rows0, g0s)
            scatter(2 * k, rows0, s0s)
            wait_s(rows1, s1s)
            gather(k, 1, rows1, g1s)
            wait_g(rows1, g1s)
            scatter(2 * k + 1, rows1, s1s)
            wait_s(rows0, s0s)
            gather(k + 1, 0, rows0, g0s)
            return carry

        lax.fori_loop(1, GROUPS_P // 2 - 1, body, 0)
        k = GROUPS_P // 2 - 1
        wait_g(rows0, g0s)
        scatter(2 * k, rows0, s0s)
        wait_s(rows1, s1s)
        gather(k, 1, rows1, g1s)
        wait_g(rows1, g1s)
        scatter(2 * k + 1, rows1, s1s)
        wait_s(rows0, s0s)
        wait_s(rows1, s1s)
        plsc.subcore_barrier()
        pltpu.sync_copy(acc.at[pl.ds(r0, RPT)],
                        out_hbm.at[c, pl.ds(r0, RPT)])

    return prop


def _sc_deg():
    mesh = plsc.VectorSubcoreMesh(core_axis_name="c", subcore_axis_name="s")

    @functools.partial(
        pl.kernel,
        out_type=jax.ShapeDtypeStruct((NCORES, N_PAD, 16), jnp.float32),
        mesh=mesh,
        compiler_params=pltpu.CompilerParams(use_tc_tiling_on_sc=False),
        scratch_types=[
            pltpu.VMEM_SHARED((N_PAD, 16), jnp.float32),
            pltpu.VMEM((GROUPS_D, BATCH), jnp.int32),
            pltpu.VMEM((BATCH, 16), jnp.float32),
            pltpu.SemaphoreType.DMA,
        ],
    )
    def deg(dst_hbm, ones_hbm, zeros_hbm, out_hbm, acc, dst_v, ones_v, sem):
        c = lax.axis_index("c")
        s = lax.axis_index("s")
        w = c * NSUB + s
        pltpu.sync_copy(dst_hbm.at[w], dst_v)
        pltpu.sync_copy(ones_hbm, ones_v)
        r0 = s * RPT
        pltpu.sync_copy(zeros_hbm.at[pl.ds(r0, RPT)], acc.at[pl.ds(r0, RPT)])
        plsc.subcore_barrier()

        def fire(g):
            pltpu.async_copy(ones_v, acc.at[dst_v.at[g]], sem, add=True)

        def drain():
            pltpu.make_async_copy(ones_v, acc.at[dst_v.at[0]], sem).wait()

        for g in range(4):
            fire(g)

        def body(g, carry):
            drain()
            fire(g)
            return carry

        lax.fori_loop(4, GROUPS_D, body, 0)
        for _ in range(4):
            drain()
        plsc.subcore_barrier()
        pltpu.sync_copy(acc.at[pl.ds(r0, RPT)],
                        out_hbm.at[c, pl.ds(r0, RPT)])

    return deg


_BM = 632


def _dinv_from_deg(degp):
    def body(p_ref, o_ref):
        deg = p_ref[0, :, 0:1] + p_ref[1, :, 0:1] + 1.0
        safe = jnp.maximum(deg, 1e-12)
        o_ref[...] = jnp.where(deg > 0, lax.rsqrt(safe), 0.0)

    return pl.pallas_call(
        body,
        grid=(N_PAD // _BM,),
        in_specs=[pl.BlockSpec((2, _BM, 16), lambda i: (0, i, 0))],
        out_specs=pl.BlockSpec((_BM, 1), lambda i: (i, 0)),
        out_shape=jax.ShapeDtypeStruct((N_PAD, 1), jnp.float32),
    )(degp)


def _dense_in(h, W, b0, dinv, d0):
    K = h.shape[1]
    dtot = W.shape[1]
    da = dtot - d0
    dh = da // 2

    def body(h_ref, w_ref, b_ref, dv_ref, z0_ref, al_ref, ar_ref):
        prod = jnp.dot(h_ref[...], w_ref[...],
                       preferred_element_type=jnp.float32)
        z0_ref[...] = prod[:, :d0] + b_ref[...]
        av = prod[:, d0:] * dv_ref[...]
        al_ref[...] = av[:, :dh]
        ar_ref[...] = av[:, dh:]

    return pl.pallas_call(
        body,
        grid=(N_PAD // _BM,),
        in_specs=[
            pl.BlockSpec((_BM, K), lambda i: (i, 0)),
            pl.BlockSpec((K, dtot), lambda i: (0, 0)),
            pl.BlockSpec((1, d0), lambda i: (0, 0)),
            pl.BlockSpec((_BM, 1), lambda i: (i, 0)),
        ],
        out_specs=[
            pl.BlockSpec((_BM, d0), lambda i: (i, 0)),
            pl.BlockSpec((_BM, dh), lambda i: (i, 0)),
            pl.BlockSpec((_BM, dh), lambda i: (i, 0)),
        ],
        out_shape=[
            jax.ShapeDtypeStruct((N_PAD, d0), jnp.float32),
            jax.ShapeDtypeStruct((N_PAD, dh), jnp.float32),
            jax.ShapeDtypeStruct((N_PAD, dh), jnp.float32),
        ],
    )(h, W, b0, dinv)


def _combine_mid(p, dinv, b1, dh):
    Fh = p.shape[2]
    da = 2 * Fh - dh
    dq = da // 2

    def body(p_ref, dv_ref, b_ref, o1_ref, gl_ref, gr_ref):
        sm = jnp.concatenate([p_ref[0], p_ref[1]], axis=1)
        dv = dv_ref[...]
        o1_ref[...] = sm[:, :dh] * dv + b_ref[...]
        gv = sm[:, dh:] * (dv * dv)
        gl_ref[...] = gv[:, :dq]
        gr_ref[...] = gv[:, dq:]

    return pl.pallas_call(
        body,
        grid=(N_PAD // _BM,),
        in_specs=[
            pl.BlockSpec((2, _BM, Fh), lambda i: (0, i, 0)),
            pl.BlockSpec((_BM, 1), lambda i: (i, 0)),
            pl.BlockSpec((1, dh), lambda i: (0, 0)),
        ],
        out_specs=[
            pl.BlockSpec((_BM, dh), lambda i: (i, 0)),
            pl.BlockSpec((_BM, dq), lambda i: (i, 0)),
            pl.BlockSpec((_BM, dq), lambda i: (i, 0)),
        ],
        out_shape=[
            jax.ShapeDtypeStruct((N_PAD, dh), jnp.float32),
            jax.ShapeDtypeStruct((N_PAD, dq), jnp.float32),
            jax.ShapeDtypeStruct((N_PAD, dq), jnp.float32),
        ],
    )(p, dinv, b1)


def _combine_out(z0, out1, q, dinv, b2):
    d0 = z0.shape[1]
    d1 = out1.shape[1]
    d2 = 2 * q.shape[2]

    def body(z0_ref, o1_ref, q_ref, dv_ref, b_ref, h_ref):
        qs = jnp.concatenate([q_ref[0], q_ref[1]], axis=1)
        o2 = qs * dv_ref[...] + b_ref[...]
        h_ref[...] = jnp.concatenate([z0_ref[...], o1_ref[...], o2], axis=1)

    return pl.pallas_call(
        body,
        grid=(N_PAD // _BM,),
        in_specs=[
            pl.BlockSpec((_BM, d0), lambda i: (i, 0)),
            pl.BlockSpec((_BM, d1), lambda i: (i, 0)),
            pl.BlockSpec((2, _BM, d2 // 2), lambda i: (0, i, 0)),
            pl.BlockSpec((_BM, 1), lambda i: (i, 0)),
            pl.BlockSpec((1, d2), lambda i: (0, 0)),
        ],
        out_specs=pl.BlockSpec((_BM, d0 + d1 + d2), lambda i: (i, 0)),
        out_shape=jax.ShapeDtypeStruct((N_PAD, d0 + d1 + d2), jnp.float32),
    )(z0, out1, q, dinv, b2)


def _final_out(z0, q, dinv, b1):
    d0 = z0.shape[1]
    d1 = 2 * q.shape[2]

    def body(z0_ref, q_ref, dv_ref, b_ref, h_ref):
        qs = jnp.concatenate([q_ref[0], q_ref[1]], axis=1)
        o1 = qs * dv_ref[...] + b_ref[...]
        h_ref[...] = jnp.concatenate([z0_ref[...], o1], axis=1)

    return pl.pallas_call(
        body,
        grid=(N_PAD // _BM,),
        in_specs=[
            pl.BlockSpec((_BM, d0), lambda i: (i, 0)),
            pl.BlockSpec((2, _BM, d1 // 2), lambda i: (0, i, 0)),
            pl.BlockSpec((_BM, 1), lambda i: (i, 0)),
            pl.BlockSpec((1, d1), lambda i: (0, 0)),
        ],
        out_specs=pl.BlockSpec((_BM, d0 + d1), lambda i: (i, 0)),
        out_shape=jax.ShapeDtypeStruct((N_PAD, d0 + d1), jnp.float32),
    )(z0, q, dinv, b1)


def kernel(x, edge_index, conv1_W, conv1_b, block_W, block_b, conv3_W, conv3_b):
    f32 = jnp.float32

    xp = jnp.pad(x, ((0, N_PAD - N_NODES), (0, 0)))
    npad = E_PAD - N_EDGES
    pad_ids = (jnp.arange(npad, dtype=jnp.int32) % PAD_SPREAD) + N_NODES
    src_flat = jnp.concatenate([edge_index[0], pad_ids])
    dst_flat = jnp.concatenate([edge_index[1], pad_ids])
    srcp = src_flat.reshape(NSUB, GROUPS_P // 2, 2 * BATCH)
    dstp = dst_flat.reshape(NSUB, GROUPS_P, BATCH)
    dstd = dst_flat.reshape(NCORES * NSUB, GROUPS_D, BATCH)

    ones16 = jnp.ones((BATCH, 16), f32)
    zeros16 = jnp.zeros((N_PAD, 16), f32)

    degp = _sc_deg()(dstd, ones16, zeros16)
    dinv = _dinv_from_deg(degp)

    prop64 = _sc_prop(64)
    prop32 = _sc_prop(32)
    prop16 = _sc_prop(16)

    def mixhop3(h, Wcat, b0, b1, b2):
        z0, aL, aR = _dense_in(h, Wcat, b0, dinv, 64)
        p = prop64(aL, aR, srcp, dstp)
        out1, gL, gR = _combine_mid(p, dinv, b1, 64)
        q = prop32(gL, gR, srcp, dstp)
        return _combine_out(z0, out1, q, dinv, b2)

    W1 = jnp.concatenate([conv1_W[0], conv1_W[1], conv1_W[2]], axis=1)
    h = mixhop3(xp, W1, conv1_b[0][None], conv1_b[1][None], conv1_b[2][None])

    for i in range(2):
        Wm = jnp.concatenate([block_W[i, 0], block_W[i, 1], block_W[i, 2]],
                             axis=1)
        h = mixhop3(h, Wm, block_b[i, 0][None], block_b[i, 1][None],
                    block_b[i, 2][None])

    W3 = jnp.concatenate([conv3_W[0], conv3_W[1]], axis=1)
    z0, aL, aR = _dense_in(h, W3, conv3_b[0][None], dinv, 32)
    q = prop16(aL, aR, srcp, dstp)
    out = _final_out(z0, q, dinv, conv3_b[1][None])
    return out[:N_NODES]


# speedup vs baseline: 1.4732x; 1.0700x over previous
import functools

import jax
import jax.numpy as jnp
from jax import lax
from jax.experimental import pallas as pl
from jax.experimental.pallas import tpu as pltpu
from jax.experimental.pallas import tpu_sc as plsc

N_NODES = 10000
N_PAD = 10112
N_EDGES = 320000
NCORES = 2
NSUB = 16
BATCH = 64
BP = 32
GROUPS_P = 640
GROUPS_D = 160
E_PAD = NSUB * GROUPS_P * BP
RPT = N_PAD // NSUB
PAD_SPREAD = 112


def _sc_prop(HALF):
    mesh = plsc.VectorSubcoreMesh(core_axis_name="c", subcore_axis_name="s")

    @functools.partial(
        pl.kernel,
        out_type=jax.ShapeDtypeStruct((NCORES, N_PAD, HALF), jnp.float32),
        mesh=mesh,
        compiler_params=pltpu.CompilerParams(use_tc_tiling_on_sc=False),
        scratch_types=[
            pltpu.VMEM_SHARED((N_PAD, HALF), jnp.float32),
            pltpu.VMEM_SHARED((N_PAD, HALF), jnp.float32),
            pltpu.VMEM((GROUPS_P // 4, 4 * BP), jnp.int32),
            pltpu.VMEM((GROUPS_P, BP), jnp.int32),
            pltpu.VMEM((BP, HALF), jnp.float32),
            pltpu.VMEM((BP, HALF), jnp.float32),
            pltpu.VMEM((BP, HALF), jnp.float32),
            pltpu.VMEM((BP, HALF), jnp.float32),
            pltpu.SemaphoreType.DMA,
            pltpu.SemaphoreType.DMA,
            pltpu.SemaphoreType.DMA,
            pltpu.SemaphoreType.DMA,
            pltpu.SemaphoreType.DMA,
            pltpu.SemaphoreType.DMA,
            pltpu.SemaphoreType.DMA,
            pltpu.SemaphoreType.DMA,
        ],
    )
    def prop(zl_hbm, zr_hbm, src_hbm, dst_hbm, out_hbm,
             zsp, acc, src_v, dst_v, r0, r1, r2, r3,
             gs0, gs1, gs2, gs3, ss0, ss1, ss2, ss3):
        rows = [r0, r1, r2, r3]
        gs = [gs0, gs1, gs2, gs3]
        ss = [ss0, ss1, ss2, ss3]
        c = lax.axis_index("c")
        s = lax.axis_index("s")
        pltpu.sync_copy(src_hbm.at[s], src_v)
        pltpu.sync_copy(dst_hbm.at[s], dst_v)
        r0 = s * RPT

        @pl.when(c == 0)
        def _():
            pltpu.sync_copy(zl_hbm.at[pl.ds(r0, RPT)], zsp.at[pl.ds(r0, RPT)])
            pltpu.sync_copy(zl_hbm.at[pl.ds(r0, RPT)], acc.at[pl.ds(r0, RPT)])

        @pl.when(c != 0)
        def _():
            pltpu.sync_copy(zr_hbm.at[pl.ds(r0, RPT)], zsp.at[pl.ds(r0, RPT)])
            pltpu.sync_copy(zr_hbm.at[pl.ds(r0, RPT)], acc.at[pl.ds(r0, RPT)])

        plsc.subcore_barrier()

        def gather(row, q, j):
            pltpu.async_copy(
                zsp.at[src_v.at[row, pl.ds(q * BP, BP)]], rows[j], gs[j])

        def wait_g(j):
            pltpu.make_async_copy(
                zsp.at[src_v.at[0, pl.ds(0, BP)]], rows[j], gs[j]).wait()

        def scatter(i, j):
            pltpu.async_copy(rows[j], acc.at[dst_v.at[i]], ss[j], add=True)

        def wait_s(j):
            pltpu.make_async_copy(rows[j], acc.at[dst_v.at[0]], ss[j]).wait()

        gather(0, 0, 0)
        gather(0, 1, 1)
        wait_g(0); scatter(0, 0); gather(0, 2, 2)
        wait_g(1); scatter(1, 1); gather(0, 3, 3)
        wait_g(2); scatter(2, 2); wait_s(0); gather(1, 0, 0)
        wait_g(3); scatter(3, 3); wait_s(1); gather(1, 1, 1)

        def body(k, carry):
            i = 4 * k
            wait_g(0); scatter(i, 0);     wait_s(2); gather(k, 2, 2)
            wait_g(1); scatter(i + 1, 1); wait_s(3); gather(k, 3, 3)
            wait_g(2); scatter(i + 2, 2); wait_s(0); gather(k + 1, 0, 0)
            wait_g(3); scatter(i + 3, 3); wait_s(1); gather(k + 1, 1, 1)
            return carry

        lax.fori_loop(1, GROUPS_P // 4 - 1, body, 0)
        i = GROUPS_P - 4
        kk = GROUPS_P // 4 - 1
        wait_g(0); scatter(i, 0);     wait_s(2); gather(kk, 2, 2)
        wait_g(1); scatter(i + 1, 1); wait_s(3); gather(kk, 3, 3)
        wait_g(2); scatter(i + 2, 2)
        wait_g(3); scatter(i + 3, 3)
        wait_s(0); wait_s(1); wait_s(2); wait_s(3)
        plsc.subcore_barrier()
        pltpu.sync_copy(acc.at[pl.ds(r0, RPT)],
                        out_hbm.at[c, pl.ds(r0, RPT)])

    return prop


def _sc_deg():
    mesh = plsc.VectorSubcoreMesh(core_axis_name="c", subcore_axis_name="s")

    @functools.partial(
        pl.kernel,
        out_type=jax.ShapeDtypeStruct((NCORES, N_PAD, 16), jnp.float32),
        mesh=mesh,
        compiler_params=pltpu.CompilerParams(use_tc_tiling_on_sc=False),
        scratch_types=[
            pltpu.VMEM_SHARED((N_PAD, 16), jnp.float32),
            pltpu.VMEM((GROUPS_D, BATCH), jnp.int32),
            pltpu.VMEM((BATCH, 16), jnp.float32),
            pltpu.SemaphoreType.DMA,
        ],
    )
    def deg(dst_hbm, ones_hbm, zeros_hbm, out_hbm, acc, dst_v, ones_v, sem):
        c = lax.axis_index("c")
        s = lax.axis_index("s")
        w = c * NSUB + s
        pltpu.sync_copy(dst_hbm.at[w], dst_v)
        pltpu.sync_copy(ones_hbm, ones_v)
        r0 = s * RPT
        pltpu.sync_copy(zeros_hbm.at[pl.ds(r0, RPT)], acc.at[pl.ds(r0, RPT)])
        plsc.subcore_barrier()

        def fire(g):
            pltpu.async_copy(ones_v, acc.at[dst_v.at[g]], sem, add=True)

        def drain():
            pltpu.make_async_copy(ones_v, acc.at[dst_v.at[0]], sem).wait()

        for g in range(4):
            fire(g)

        def body(g, carry):
            drain()
            fire(g)
            return carry

        lax.fori_loop(4, GROUPS_D, body, 0)
        for _ in range(4):
            drain()
        plsc.subcore_barrier()
        pltpu.sync_copy(acc.at[pl.ds(r0, RPT)],
                        out_hbm.at[c, pl.ds(r0, RPT)])

    return deg


_BM = 632


def _dinv_from_deg(degp):
    def body(p_ref, o_ref):
        deg = p_ref[0, :, 0:1] + p_ref[1, :, 0:1] + 1.0
        safe = jnp.maximum(deg, 1e-12)
        o_ref[...] = jnp.where(deg > 0, lax.rsqrt(safe), 0.0)

    return pl.pallas_call(
        body,
        grid=(N_PAD // _BM,),
        in_specs=[pl.BlockSpec((2, _BM, 16), lambda i: (0, i, 0))],
        out_specs=pl.BlockSpec((_BM, 1), lambda i: (i, 0)),
        out_shape=jax.ShapeDtypeStruct((N_PAD, 1), jnp.float32),
    )(degp)


def _dense_in(h, W, b0, dinv, d0):
    K = h.shape[1]
    dtot = W.shape[1]
    da = dtot - d0
    dh = da // 2

    def body(h_ref, w_ref, b_ref, dv_ref, z0_ref, al_ref, ar_ref):
        prod = jnp.dot(h_ref[...], w_ref[...],
                       preferred_element_type=jnp.float32)
        z0_ref[...] = prod[:, :d0] + b_ref[...]
        av = prod[:, d0:] * dv_ref[...]
        al_ref[...] = av[:, :dh]
        ar_ref[...] = av[:, dh:]

    return pl.pallas_call(
        body,
        grid=(N_PAD // _BM,),
        in_specs=[
            pl.BlockSpec((_BM, K), lambda i: (i, 0)),
            pl.BlockSpec((K, dtot), lambda i: (0, 0)),
            pl.BlockSpec((1, d0), lambda i: (0, 0)),
            pl.BlockSpec((_BM, 1), lambda i: (i, 0)),
        ],
        out_specs=[
            pl.BlockSpec((_BM, d0), lambda i: (i, 0)),
            pl.BlockSpec((_BM, dh), lambda i: (i, 0)),
            pl.BlockSpec((_BM, dh), lambda i: (i, 0)),
        ],
        out_shape=[
            jax.ShapeDtypeStruct((N_PAD, d0), jnp.float32),
            jax.ShapeDtypeStruct((N_PAD, dh), jnp.float32),
            jax.ShapeDtypeStruct((N_PAD, dh), jnp.float32),
        ],
    )(h, W, b0, dinv)


def _combine_mid(p, dinv, b1, dh):
    Fh = p.shape[2]
    da = 2 * Fh - dh
    dq = da // 2

    def body(p_ref, dv_ref, b_ref, o1_ref, gl_ref, gr_ref):
        sm = jnp.concatenate([p_ref[0], p_ref[1]], axis=1)
        dv = dv_ref[...]
        o1_ref[...] = sm[:, :dh] * dv + b_ref[...]
        gv = sm[:, dh:] * (dv * dv)
        gl_ref[...] = gv[:, :dq]
        gr_ref[...] = gv[:, dq:]

    return pl.pallas_call(
        body,
        grid=(N_PAD // _BM,),
        in_specs=[
            pl.BlockSpec((2, _BM, Fh), lambda i: (0, i, 0)),
            pl.BlockSpec((_BM, 1), lambda i: (i, 0)),
            pl.BlockSpec((1, dh), lambda i: (0, 0)),
        ],
        out_specs=[
            pl.BlockSpec((_BM, dh), lambda i: (i, 0)),
            pl.BlockSpec((_BM, dq), lambda i: (i, 0)),
            pl.BlockSpec((_BM, dq), lambda i: (i, 0)),
        ],
        out_shape=[
            jax.ShapeDtypeStruct((N_PAD, dh), jnp.float32),
            jax.ShapeDtypeStruct((N_PAD, dq), jnp.float32),
            jax.ShapeDtypeStruct((N_PAD, dq), jnp.float32),
        ],
    )(p, dinv, b1)


def _combine_out(z0, out1, q, dinv, b2):
    d0 = z0.shape[1]
    d1 = out1.shape[1]
    d2 = 2 * q.shape[2]

    def body(z0_ref, o1_ref, q_ref, dv_ref, b_ref, h_ref):
        qs = jnp.concatenate([q_ref[0], q_ref[1]], axis=1)
        o2 = qs * dv_ref[...] + b_ref[...]
        h_ref[...] = jnp.concatenate([z0_ref[...], o1_ref[...], o2], axis=1)

    return pl.pallas_call(
        body,
        grid=(N_PAD // _BM,),
        in_specs=[
            pl.BlockSpec((_BM, d0), lambda i: (i, 0)),
            pl.BlockSpec((_BM, d1), lambda i: (i, 0)),
            pl.BlockSpec((2, _BM, d2 // 2), lambda i: (0, i, 0)),
            pl.BlockSpec((_BM, 1), lambda i: (i, 0)),
            pl.BlockSpec((1, d2), lambda i: (0, 0)),
        ],
        out_specs=pl.BlockSpec((_BM, d0 + d1 + d2), lambda i: (i, 0)),
        out_shape=jax.ShapeDtypeStruct((N_PAD, d0 + d1 + d2), jnp.float32),
    )(z0, out1, q, dinv, b2)


def _final_out(z0, q, dinv, b1):
    d0 = z0.shape[1]
    d1 = 2 * q.shape[2]

    def body(z0_ref, q_ref, dv_ref, b_ref, h_ref):
        qs = jnp.concatenate([q_ref[0], q_ref[1]], axis=1)
        o1 = qs * dv_ref[...] + b_ref[...]
        h_ref[...] = jnp.concatenate([z0_ref[...], o1], axis=1)

    return pl.pallas_call(
        body,
        grid=(N_PAD // _BM,),
        in_specs=[
            pl.BlockSpec((_BM, d0), lambda i: (i, 0)),
            pl.BlockSpec((2, _BM, d1 // 2), lambda i: (0, i, 0)),
            pl.BlockSpec((_BM, 1), lambda i: (i, 0)),
            pl.BlockSpec((1, d1), lambda i: (0, 0)),
        ],
        out_specs=pl.BlockSpec((_BM, d0 + d1), lambda i: (i, 0)),
        out_shape=jax.ShapeDtypeStruct((N_PAD, d0 + d1), jnp.float32),
    )(z0, q, dinv, b1)


def kernel(x, edge_index, conv1_W, conv1_b, block_W, block_b, conv3_W, conv3_b):
    f32 = jnp.float32

    xp = jnp.pad(x, ((0, N_PAD - N_NODES), (0, 0)))
    npad = E_PAD - N_EDGES
    pad_ids = (jnp.arange(npad, dtype=jnp.int32) % PAD_SPREAD) + N_NODES
    src_flat = jnp.concatenate([edge_index[0], pad_ids])
    dst_flat = jnp.concatenate([edge_index[1], pad_ids])
    srcp = src_flat.reshape(NSUB, GROUPS_P // 4, 4 * BP)
    dstp = dst_flat.reshape(NSUB, GROUPS_P, BP)
    dstd = dst_flat.reshape(NCORES * NSUB, GROUPS_D, BATCH)

    ones16 = jnp.ones((BATCH, 16), f32)
    zeros16 = jnp.zeros((N_PAD, 16), f32)

    degp = _sc_deg()(dstd, ones16, zeros16)
    dinv = _dinv_from_deg(degp)

    prop64 = _sc_prop(64)
    prop32 = _sc_prop(32)
    prop16 = _sc_prop(16)

    def mixhop3(h, Wcat, b0, b1, b2):
        z0, aL, aR = _dense_in(h, Wcat, b0, dinv, 64)
        p = prop64(aL, aR, srcp, dstp)
        out1, gL, gR = _combine_mid(p, dinv, b1, 64)
        q = prop32(gL, gR, srcp, dstp)
        return _combine_out(z0, out1, q, dinv, b2)

    W1 = jnp.concatenate([conv1_W[0], conv1_W[1], conv1_W[2]], axis=1)
    h = mixhop3(xp, W1, conv1_b[0][None], conv1_b[1][None], conv1_b[2][None])

    for i in range(2):
        Wm = jnp.concatenate([block_W[i, 0], block_W[i, 1], block_W[i, 2]],
                             axis=1)
        h = mixhop3(h, Wm, block_b[i, 0][None], block_b[i, 1][None],
                    block_b[i, 2][None])

    W3 = jnp.concatenate([conv3_W[0], conv3_W[1]], axis=1)
    z0, aL, aR = _dense_in(h, W3, conv3_b[0][None], dinv, 32)
    q = prop16(aL, aR, srcp, dstp)
    out = _final_out(z0, q, dinv, conv3_b[1][None])
    return out[:N_NODES]


# speedup vs baseline: 1.5315x; 1.0396x over previous
import functools

import jax
import jax.numpy as jnp
from jax import lax
from jax.experimental import pallas as pl
from jax.experimental.pallas import tpu as pltpu
from jax.experimental.pallas import tpu_sc as plsc

N_NODES = 10000
N_PAD = 10112
N_EDGES = 320000
NCORES = 2
NSUB = 16
BATCH = 64
BP = 16
GROUPS_P = 1280
GROUPS_D = 160
E_PAD = NSUB * GROUPS_P * BP
RPT = N_PAD // NSUB
PAD_SPREAD = 112


def _sc_prop(HALF):
    mesh = plsc.VectorSubcoreMesh(core_axis_name="c", subcore_axis_name="s")

    @functools.partial(
        pl.kernel,
        out_type=jax.ShapeDtypeStruct((NCORES, N_PAD, HALF), jnp.float32),
        mesh=mesh,
        compiler_params=pltpu.CompilerParams(use_tc_tiling_on_sc=False),
        scratch_types=[
            pltpu.VMEM_SHARED((N_PAD, HALF), jnp.float32),
            pltpu.VMEM_SHARED((N_PAD, HALF), jnp.float32),
            pltpu.VMEM((GROUPS_P // 8, 8 * BP), jnp.int32),
            pltpu.VMEM((GROUPS_P, BP), jnp.int32),
        ] + [pltpu.VMEM((BP, HALF), jnp.float32)] * 8
          + [pltpu.SemaphoreType.DMA] * 16,
    )
    def prop(zl_hbm, zr_hbm, src_hbm, dst_hbm, out_hbm,
             zsp, acc, src_v, dst_v, *bufs):
        rows = list(bufs[0:8])
        gs = list(bufs[8:16])
        ss = list(bufs[16:24])
        c = lax.axis_index("c")
        s = lax.axis_index("s")
        pltpu.sync_copy(src_hbm.at[s], src_v)
        pltpu.sync_copy(dst_hbm.at[s], dst_v)
        r0 = s * RPT

        @pl.when(c == 0)
        def _():
            pltpu.sync_copy(zl_hbm.at[pl.ds(r0, RPT)], zsp.at[pl.ds(r0, RPT)])
            pltpu.sync_copy(zl_hbm.at[pl.ds(r0, RPT)], acc.at[pl.ds(r0, RPT)])

        @pl.when(c != 0)
        def _():
            pltpu.sync_copy(zr_hbm.at[pl.ds(r0, RPT)], zsp.at[pl.ds(r0, RPT)])
            pltpu.sync_copy(zr_hbm.at[pl.ds(r0, RPT)], acc.at[pl.ds(r0, RPT)])

        plsc.subcore_barrier()

        def gather(row, q, j):
            pltpu.async_copy(
                zsp.at[src_v.at[row, pl.ds(q * BP, BP)]], rows[j], gs[j])

        def wait_g(j):
            pltpu.make_async_copy(
                zsp.at[src_v.at[0, pl.ds(0, BP)]], rows[j], gs[j]).wait()

        def scatter(i, j):
            pltpu.async_copy(rows[j], acc.at[dst_v.at[i]], ss[j], add=True)

        def wait_s(j):
            pltpu.make_async_copy(rows[j], acc.at[dst_v.at[0]], ss[j]).wait()

        for j in range(4):
            gather(0, j, j)
        for j in range(4):
            wait_g(j); scatter(j, j); gather(0, 4 + j, 4 + j)
        for j in range(4):
            wait_g(4 + j); scatter(4 + j, 4 + j); wait_s(j); gather(1, j, j)

        def body(k, carry):
            i = 8 * k
            wait_g(0); scatter(i, 0);     wait_s(4); gather(k, 4, 4)
            wait_g(1); scatter(i + 1, 1); wait_s(5); gather(k, 5, 5)
            wait_g(2); scatter(i + 2, 2); wait_s(6); gather(k, 6, 6)
            wait_g(3); scatter(i + 3, 3); wait_s(7); gather(k, 7, 7)
            wait_g(4); scatter(i + 4, 4); wait_s(0); gather(k + 1, 0, 0)
            wait_g(5); scatter(i + 5, 5); wait_s(1); gather(k + 1, 1, 1)
            wait_g(6); scatter(i + 6, 6); wait_s(2); gather(k + 1, 2, 2)
            wait_g(7); scatter(i + 7, 7); wait_s(3); gather(k + 1, 3, 3)
            return carry

        lax.fori_loop(1, GROUPS_P // 8 - 1, body, 0)
        i = GROUPS_P - 8
        kk = GROUPS_P // 8 - 1
        wait_g(0); scatter(i, 0);     wait_s(4); gather(kk, 4, 4)
        wait_g(1); scatter(i + 1, 1); wait_s(5); gather(kk, 5, 5)
        wait_g(2); scatter(i + 2, 2); wait_s(6); gather(kk, 6, 6)
        wait_g(3); scatter(i + 3, 3); wait_s(7); gather(kk, 7, 7)
        for j in range(4):
            wait_g(4 + j); scatter(i + 4 + j, 4 + j)
        for j in range(8):
            wait_s(j)
        plsc.subcore_barrier()
        pltpu.sync_copy(acc.at[pl.ds(r0, RPT)],
                        out_hbm.at[c, pl.ds(r0, RPT)])

    return prop


def _sc_deg():
    mesh = plsc.VectorSubcoreMesh(core_axis_name="c", subcore_axis_name="s")

    @functools.partial(
        pl.kernel,
        out_type=jax.ShapeDtypeStruct((NCORES, N_PAD, 16), jnp.float32),
        mesh=mesh,
        compiler_params=pltpu.CompilerParams(use_tc_tiling_on_sc=False),
        scratch_types=[
            pltpu.VMEM_SHARED((N_PAD, 16), jnp.float32),
            pltpu.VMEM((GROUPS_D, BATCH), jnp.int32),
            pltpu.VMEM((BATCH, 16), jnp.float32),
            pltpu.SemaphoreType.DMA,
        ],
    )
    def deg(dst_hbm, ones_hbm, zeros_hbm, out_hbm, acc, dst_v, ones_v, sem):
        c = lax.axis_index("c")
        s = lax.axis_index("s")
        w = c * NSUB + s
        pltpu.sync_copy(dst_hbm.at[w], dst_v)
        pltpu.sync_copy(ones_hbm, ones_v)
        r0 = s * RPT
        pltpu.sync_copy(zeros_hbm.at[pl.ds(r0, RPT)], acc.at[pl.ds(r0, RPT)])
        plsc.subcore_barrier()

        def fire(g):
            pltpu.async_copy(ones_v, acc.at[dst_v.at[g]], sem, add=True)

        def drain():
            pltpu.make_async_copy(ones_v, acc.at[dst_v.at[0]], sem).wait()

        for g in range(4):
            fire(g)

        def body(g, carry):
            drain()
            fire(g)
            return carry

        lax.fori_loop(4, GROUPS_D, body, 0)
        for _ in range(4):
            drain()
        plsc.subcore_barrier()
        pltpu.sync_copy(acc.at[pl.ds(r0, RPT)],
                        out_hbm.at[c, pl.ds(r0, RPT)])

    return deg


_BM = 632


def _dinv_from_deg(degp):
    def body(p_ref, o_ref):
        deg = p_ref[0, :, 0:1] + p_ref[1, :, 0:1] + 1.0
        safe = jnp.maximum(deg, 1e-12)
        o_ref[...] = jnp.where(deg > 0, lax.rsqrt(safe), 0.0)

    return pl.pallas_call(
        body,
        grid=(N_PAD // _BM,),
        in_specs=[pl.BlockSpec((2, _BM, 16), lambda i: (0, i, 0))],
        out_specs=pl.BlockSpec((_BM, 1), lambda i: (i, 0)),
        out_shape=jax.ShapeDtypeStruct((N_PAD, 1), jnp.float32),
    )(degp)


def _dense_in(h, W, b0, dinv, d0):
    K = h.shape[1]
    dtot = W.shape[1]
    da = dtot - d0
    dh = da // 2

    def body(h_ref, w_ref, b_ref, dv_ref, z0_ref, al_ref, ar_ref):
        prod = jnp.dot(h_ref[...], w_ref[...],
                       preferred_element_type=jnp.float32)
        z0_ref[...] = prod[:, :d0] + b_ref[...]
        av = prod[:, d0:] * dv_ref[...]
        al_ref[...] = av[:, :dh]
        ar_ref[...] = av[:, dh:]

    return pl.pallas_call(
        body,
        grid=(N_PAD // _BM,),
        in_specs=[
            pl.BlockSpec((_BM, K), lambda i: (i, 0)),
            pl.BlockSpec((K, dtot), lambda i: (0, 0)),
            pl.BlockSpec((1, d0), lambda i: (0, 0)),
            pl.BlockSpec((_BM, 1), lambda i: (i, 0)),
        ],
        out_specs=[
            pl.BlockSpec((_BM, d0), lambda i: (i, 0)),
            pl.BlockSpec((_BM, dh), lambda i: (i, 0)),
            pl.BlockSpec((_BM, dh), lambda i: (i, 0)),
        ],
        out_shape=[
            jax.ShapeDtypeStruct((N_PAD, d0), jnp.float32),
            jax.ShapeDtypeStruct((N_PAD, dh), jnp.float32),
            jax.ShapeDtypeStruct((N_PAD, dh), jnp.float32),
        ],
    )(h, W, b0, dinv)


def _combine_mid(p, dinv, b1, dh):
    Fh = p.shape[2]
    da = 2 * Fh - dh
    dq = da // 2

    def body(p_ref, dv_ref, b_ref, o1_ref, gl_ref, gr_ref):
        sm = jnp.concatenate([p_ref[0], p_ref[1]], axis=1)
        dv = dv_ref[...]
        o1_ref[...] = sm[:, :dh] * dv + b_ref[...]
        gv = sm[:, dh:] * (dv * dv)
        gl_ref[...] = gv[:, :dq]
        gr_ref[...] = gv[:, dq:]

    return pl.pallas_call(
        body,
        grid=(N_PAD // _BM,),
        in_specs=[
            pl.BlockSpec((2, _BM, Fh), lambda i: (0, i, 0)),
            pl.BlockSpec((_BM, 1), lambda i: (i, 0)),
            pl.BlockSpec((1, dh), lambda i: (0, 0)),
        ],
        out_specs=[
            pl.BlockSpec((_BM, dh), lambda i: (i, 0)),
            pl.BlockSpec((_BM, dq), lambda i: (i, 0)),
            pl.BlockSpec((_BM, dq), lambda i: (i, 0)),
        ],
        out_shape=[
            jax.ShapeDtypeStruct((N_PAD, dh), jnp.float32),
            jax.ShapeDtypeStruct((N_PAD, dq), jnp.float32),
            jax.ShapeDtypeStruct((N_PAD, dq), jnp.float32),
        ],
    )(p, dinv, b1)


def _combine_out(z0, out1, q, dinv, b2):
    d0 = z0.shape[1]
    d1 = out1.shape[1]
    d2 = 2 * q.shape[2]

    def body(z0_ref, o1_ref, q_ref, dv_ref, b_ref, h_ref):
        qs = jnp.concatenate([q_ref[0], q_ref[1]], axis=1)
        o2 = qs * dv_ref[...] + b_ref[...]
        h_ref[...] = jnp.concatenate([z0_ref[...], o1_ref[...], o2], axis=1)

    return pl.pallas_call(
        body,
        grid=(N_PAD // _BM,),
        in_specs=[
            pl.BlockSpec((_BM, d0), lambda i: (i, 0)),
            pl.BlockSpec((_BM, d1), lambda i: (i, 0)),
            pl.BlockSpec((2, _BM, d2 // 2), lambda i: (0, i, 0)),
            pl.BlockSpec((_BM, 1), lambda i: (i, 0)),
            pl.BlockSpec((1, d2), lambda i: (0, 0)),
        ],
        out_specs=pl.BlockSpec((_BM, d0 + d1 + d2), lambda i: (i, 0)),
        out_shape=jax.ShapeDtypeStruct((N_PAD, d0 + d1 + d2), jnp.float32),
    )(z0, out1, q, dinv, b2)


def _final_out(z0, q, dinv, b1):
    d0 = z0.shape[1]
    d1 = 2 * q.shape[2]

    def body(z0_ref, q_ref, dv_ref, b_ref, h_ref):
        qs = jnp.concatenate([q_ref[0], q_ref[1]], axis=1)
        o1 = qs * dv_ref[...] + b_ref[...]
        h_ref[...] = jnp.concatenate([z0_ref[...], o1], axis=1)

    return pl.pallas_call(
        body,
        grid=(N_PAD // _BM,),
        in_specs=[
            pl.BlockSpec((_BM, d0), lambda i: (i, 0)),
            pl.BlockSpec((2, _BM, d1 // 2), lambda i: (0, i, 0)),
            pl.BlockSpec((_BM, 1), lambda i: (i, 0)),
            pl.BlockSpec((1, d1), lambda i: (0, 0)),
        ],
        out_specs=pl.BlockSpec((_BM, d0 + d1), lambda i: (i, 0)),
        out_shape=jax.ShapeDtypeStruct((N_PAD, d0 + d1), jnp.float32),
    )(z0, q, dinv, b1)


def kernel(x, edge_index, conv1_W, conv1_b, block_W, block_b, conv3_W, conv3_b):
    f32 = jnp.float32

    xp = jnp.pad(x, ((0, N_PAD - N_NODES), (0, 0)))
    npad = E_PAD - N_EDGES
    pad_ids = (jnp.arange(npad, dtype=jnp.int32) % PAD_SPREAD) + N_NODES
    src_flat = jnp.concatenate([edge_index[0], pad_ids])
    dst_flat = jnp.concatenate([edge_index[1], pad_ids])
    srcp = src_flat.reshape(NSUB, GROUPS_P // 8, 8 * BP)
    dstp = dst_flat.reshape(NSUB, GROUPS_P, BP)
    dstd = dst_flat.reshape(NCORES * NSUB, GROUPS_D, BATCH)

    ones16 = jnp.ones((BATCH, 16), f32)
    zeros16 = jnp.zeros((N_PAD, 16), f32)

    degp = _sc_deg()(dstd, ones16, zeros16)
    dinv = _dinv_from_deg(degp)

    prop64 = _sc_prop(64)
    prop32 = _sc_prop(32)
    prop16 = _sc_prop(16)

    def mixhop3(h, Wcat, b0, b1, b2):
        z0, aL, aR = _dense_in(h, Wcat, b0, dinv, 64)
        p = prop64(aL, aR, srcp, dstp)
        out1, gL, gR = _combine_mid(p, dinv, b1, 64)
        q = prop32(gL, gR, srcp, dstp)
        return _combine_out(z0, out1, q, dinv, b2)

    W1 = jnp.concatenate([conv1_W[0], conv1_W[1], conv1_W[2]], axis=1)
    h = mixhop3(xp, W1, conv1_b[0][None], conv1_b[1][None], conv1_b[2][None])

    for i in range(2):
        Wm = jnp.concatenate([block_W[i, 0], block_W[i, 1], block_W[i, 2]],
                             axis=1)
        h = mixhop3(h, Wm, block_b[i, 0][None], block_b[i, 1][None],
                    block_b[i, 2][None])

    W3 = jnp.concatenate([conv3_W[0], conv3_W[1]], axis=1)
    z0, aL, aR = _dense_in(h, W3, conv3_b[0][None], dinv, 32)
    q = prop16(aL, aR, srcp, dstp)
    out = _final_out(z0, q, dinv, conv3_b[1][None])
    return out[:N_NODES]


# speedup vs baseline: 1.6114x; 1.0522x over previous
import functools

import jax
import jax.numpy as jnp
from jax import lax
from jax.experimental import pallas as pl
from jax.experimental.pallas import tpu as pltpu
from jax.experimental.pallas import tpu_sc as plsc

N_NODES = 10000
N_PAD = 10112
N_EDGES = 320000
NCORES = 2
NSUB = 16
BATCH = 64
BP = 16
GROUPS_P = 1280
GROUPS_D = 160
E_PAD = NSUB * GROUPS_P * BP
RPT = N_PAD // NSUB
PAD_SPREAD = 112


def _sc_prop(HALF):
    mesh = plsc.VectorSubcoreMesh(core_axis_name="c", subcore_axis_name="s")

    @functools.partial(
        pl.kernel,
        out_type=jax.ShapeDtypeStruct((NCORES, N_PAD, HALF), jnp.float32),
        mesh=mesh,
        compiler_params=pltpu.CompilerParams(use_tc_tiling_on_sc=False),
        scratch_types=[
            pltpu.VMEM_SHARED((N_PAD, HALF), jnp.float32),
            pltpu.VMEM_SHARED((N_PAD, HALF), jnp.float32),
            pltpu.VMEM((GROUPS_P // 8, 8 * BP), jnp.int32),
            pltpu.VMEM((GROUPS_P, BP), jnp.int32),
        ] + [pltpu.VMEM((BP, HALF), jnp.float32)] * 8
          + [pltpu.SemaphoreType.DMA] * 16,
    )
    def prop(zl_hbm, zr_hbm, src_hbm, dst_hbm, out_hbm,
             zsp, acc, src_v, dst_v, *bufs):
        rows = list(bufs[0:8])
        gs = list(bufs[8:16])
        ss = list(bufs[16:24])
        c = lax.axis_index("c")
        s = lax.axis_index("s")
        pltpu.sync_copy(src_hbm.at[s], src_v)
        pltpu.sync_copy(dst_hbm.at[s], dst_v)
        r0 = s * RPT

        @pl.when(c == 0)
        def _():
            pltpu.sync_copy(zl_hbm.at[pl.ds(r0, RPT)], zsp.at[pl.ds(r0, RPT)])
            pltpu.sync_copy(zl_hbm.at[pl.ds(r0, RPT)], acc.at[pl.ds(r0, RPT)])

        @pl.when(c != 0)
        def _():
            pltpu.sync_copy(zr_hbm.at[pl.ds(r0, RPT)], zsp.at[pl.ds(r0, RPT)])
            pltpu.sync_copy(zr_hbm.at[pl.ds(r0, RPT)], acc.at[pl.ds(r0, RPT)])

        plsc.subcore_barrier()

        def gather(row, q, j):
            pltpu.async_copy(
                zsp.at[src_v.at[row, pl.ds(q * BP, BP)]], rows[j], gs[j])

        def wait_g(j):
            pltpu.make_async_copy(
                zsp.at[src_v.at[0, pl.ds(0, BP)]], rows[j], gs[j]).wait()

        def scatter(i, j):
            pltpu.async_copy(rows[j], acc.at[dst_v.at[i]], ss[j], add=True)

        def wait_s(j):
            pltpu.make_async_copy(rows[j], acc.at[dst_v.at[0]], ss[j]).wait()

        for j in range(4):
            gather(0, j, j)
        for j in range(4):
            wait_g(j); scatter(j, j); gather(0, 4 + j, 4 + j)
        for j in range(4):
            wait_g(4 + j); scatter(4 + j, 4 + j); wait_s(j); gather(1, j, j)

        def body(k, carry):
            i = 8 * k
            wait_g(0); scatter(i, 0);     wait_s(4); gather(k, 4, 4)
            wait_g(1); scatter(i + 1, 1); wait_s(5); gather(k, 5, 5)
            wait_g(2); scatter(i + 2, 2); wait_s(6); gather(k, 6, 6)
            wait_g(3); scatter(i + 3, 3); wait_s(7); gather(k, 7, 7)
            wait_g(4); scatter(i + 4, 4); wait_s(0); gather(k + 1, 0, 0)
            wait_g(5); scatter(i + 5, 5); wait_s(1); gather(k + 1, 1, 1)
            wait_g(6); scatter(i + 6, 6); wait_s(2); gather(k + 1, 2, 2)
            wait_g(7); scatter(i + 7, 7); wait_s(3); gather(k + 1, 3, 3)
            return carry

        lax.fori_loop(1, GROUPS_P // 8 - 1, body, 0)
        i = GROUPS_P - 8
        kk = GROUPS_P // 8 - 1
        wait_g(0); scatter(i, 0);     wait_s(4); gather(kk, 4, 4)
        wait_g(1); scatter(i + 1, 1); wait_s(5); gather(kk, 5, 5)
        wait_g(2); scatter(i + 2, 2); wait_s(6); gather(kk, 6, 6)
        wait_g(3); scatter(i + 3, 3); wait_s(7); gather(kk, 7, 7)
        for j in range(4):
            wait_g(4 + j); scatter(i + 4 + j, 4 + j)
        for j in range(8):
            wait_s(j)
        plsc.subcore_barrier()
        pltpu.sync_copy(acc.at[pl.ds(r0, RPT)],
                        out_hbm.at[c, pl.ds(r0, RPT)])

    return prop


def _sc_deg():
    mesh = plsc.VectorSubcoreMesh(core_axis_name="c", subcore_axis_name="s")

    @functools.partial(
        pl.kernel,
        out_type=jax.ShapeDtypeStruct((NCORES, N_PAD, 16), jnp.float32),
        mesh=mesh,
        compiler_params=pltpu.CompilerParams(use_tc_tiling_on_sc=False),
        scratch_types=[
            pltpu.VMEM_SHARED((N_PAD, 16), jnp.float32),
            pltpu.VMEM((GROUPS_D, BATCH), jnp.int32),
            pltpu.VMEM((BATCH, 16), jnp.float32),
            pltpu.SemaphoreType.DMA,
        ],
    )
    def deg(dst_hbm, ones_hbm, zeros_hbm, out_hbm, acc, dst_v, ones_v, sem):
        c = lax.axis_index("c")
        s = lax.axis_index("s")
        w = c * NSUB + s
        pltpu.sync_copy(dst_hbm.at[w], dst_v)
        pltpu.sync_copy(ones_hbm, ones_v)
        r0 = s * RPT
        pltpu.sync_copy(zeros_hbm.at[pl.ds(r0, RPT)], acc.at[pl.ds(r0, RPT)])
        plsc.subcore_barrier()

        def fire(g):
            pltpu.async_copy(ones_v, acc.at[dst_v.at[g]], sem, add=True)

        def drain():
            pltpu.make_async_copy(ones_v, acc.at[dst_v.at[0]], sem).wait()

        for g in range(4):
            fire(g)

        def body(g, carry):
            drain()
            fire(g)
            return carry

        lax.fori_loop(4, GROUPS_D, body, 0)
        for _ in range(4):
            drain()
        plsc.subcore_barrier()
        pltpu.sync_copy(acc.at[pl.ds(r0, RPT)],
                        out_hbm.at[c, pl.ds(r0, RPT)])

    return deg


_BM = 632


def _dinv_from_deg(degp):
    def body(p_ref, o_ref):
        deg = p_ref[0, :, 0:1] + p_ref[1, :, 0:1] + 1.0
        safe = jnp.maximum(deg, 1e-12)
        o_ref[...] = jnp.where(deg > 0, lax.rsqrt(safe), 0.0)

    return pl.pallas_call(
        body,
        grid=(N_PAD // _BM,),
        in_specs=[pl.BlockSpec((2, _BM, 16), lambda i: (0, i, 0))],
        out_specs=pl.BlockSpec((_BM, 1), lambda i: (i, 0)),
        out_shape=jax.ShapeDtypeStruct((N_PAD, 1), jnp.float32),
    )(degp)


def _dense_in(h, W, b0, dinv, d0):
    K = h.shape[1]
    dtot = W.shape[1]
    da = dtot - d0
    dh = da // 2

    def body(h_ref, w_ref, b_ref, dv_ref, z0_ref, al_ref, ar_ref):
        prod = jnp.dot(h_ref[...], w_ref[...],
                       preferred_element_type=jnp.float32)
        z0_ref[...] = prod[:, :d0] + b_ref[...]
        av = prod[:, d0:] * dv_ref[...]
        al_ref[...] = av[:, :dh]
        ar_ref[...] = av[:, dh:]

    return pl.pallas_call(
        body,
        grid=(N_PAD // _BM,),
        in_specs=[
            pl.BlockSpec((_BM, K), lambda i: (i, 0)),
            pl.BlockSpec((K, dtot), lambda i: (0, 0)),
            pl.BlockSpec((1, d0), lambda i: (0, 0)),
            pl.BlockSpec((_BM, 1), lambda i: (i, 0)),
        ],
        out_specs=[
            pl.BlockSpec((_BM, d0), lambda i: (i, 0)),
            pl.BlockSpec((_BM, dh), lambda i: (i, 0)),
            pl.BlockSpec((_BM, dh), lambda i: (i, 0)),
        ],
        out_shape=[
            jax.ShapeDtypeStruct((N_PAD, d0), jnp.float32),
            jax.ShapeDtypeStruct((N_PAD, dh), jnp.float32),
            jax.ShapeDtypeStruct((N_PAD, dh), jnp.float32),
        ],
    )(h, W, b0, dinv)


def _combine_mid(p, dinv, b1, dh):
    Fh = p.shape[2]
    da = 2 * Fh - dh
    dq = da // 2

    def body(p_ref, dv_ref, b_ref, o1_ref, gl_ref, gr_ref):
        sm = jnp.concatenate([p_ref[0], p_ref[1]], axis=1)
        dv = dv_ref[...]
        o1_ref[...] = sm[:, :dh] * dv + b_ref[...]
        gv = sm[:, dh:] * (dv * dv)
        gl_ref[...] = gv[:, :dq]
        gr_ref[...] = gv[:, dq:]

    return pl.pallas_call(
        body,
        grid=(N_PAD // _BM,),
        in_specs=[
            pl.BlockSpec((2, _BM, Fh), lambda i: (0, i, 0)),
            pl.BlockSpec((_BM, 1), lambda i: (i, 0)),
            pl.BlockSpec((1, dh), lambda i: (0, 0)),
        ],
        out_specs=[
            pl.BlockSpec((_BM, dh), lambda i: (i, 0)),
            pl.BlockSpec((_BM, dq), lambda i: (i, 0)),
            pl.BlockSpec((_BM, dq), lambda i: (i, 0)),
        ],
        out_shape=[
            jax.ShapeDtypeStruct((N_PAD, dh), jnp.float32),
            jax.ShapeDtypeStruct((N_PAD, dq), jnp.float32),
            jax.ShapeDtypeStruct((N_PAD, dq), jnp.float32),
        ],
    )(p, dinv, b1)


def _combine_out(z0, out1, q, dinv, b2):
    d0 = z0.shape[1]
    d1 = out1.shape[1]
    d2 = 2 * q.shape[2]

    def body(z0_ref, o1_ref, q_ref, dv_ref, b_ref, h_ref):
        qs = jnp.concatenate([q_ref[0], q_ref[1]], axis=1)
        o2 = qs * dv_ref[...] + b_ref[...]
        h_ref[...] = jnp.concatenate([z0_ref[...], o1_ref[...], o2], axis=1)

    return pl.pallas_call(
        body,
        grid=(N_PAD // _BM,),
        in_specs=[
            pl.BlockSpec((_BM, d0), lambda i: (i, 0)),
            pl.BlockSpec((_BM, d1), lambda i: (i, 0)),
            pl.BlockSpec((2, _BM, d2 // 2), lambda i: (0, i, 0)),
            pl.BlockSpec((_BM, 1), lambda i: (i, 0)),
            pl.BlockSpec((1, d2), lambda i: (0, 0)),
        ],
        out_specs=pl.BlockSpec((_BM, d0 + d1 + d2), lambda i: (i, 0)),
        out_shape=jax.ShapeDtypeStruct((N_PAD, d0 + d1 + d2), jnp.float32),
    )(z0, out1, q, dinv, b2)


def _combine_dense(z0, out1, q, dinv, b2, W, b0n, d0n):
    d0 = z0.shape[1]
    d1 = out1.shape[1]
    d2 = 2 * q.shape[2]
    dtot = W.shape[1]
    dh = (dtot - d0n) // 2

    def body(z0_ref, o1_ref, q_ref, dv_ref, b2_ref, w_ref, b0_ref,
             z0n_ref, al_ref, ar_ref):
        dv = dv_ref[...]
        qs = jnp.concatenate([q_ref[0], q_ref[1]], axis=1)
        h = jnp.concatenate(
            [z0_ref[...], o1_ref[...], qs * dv + b2_ref[...]], axis=1)
        prod = jnp.dot(h, w_ref[...], preferred_element_type=jnp.float32)
        z0n_ref[...] = prod[:, :d0n] + b0_ref[...]
        av = prod[:, d0n:] * dv
        al_ref[...] = av[:, :dh]
        ar_ref[...] = av[:, dh:]

    return pl.pallas_call(
        body,
        grid=(N_PAD // _BM,),
        in_specs=[
            pl.BlockSpec((_BM, d0), lambda i: (i, 0)),
            pl.BlockSpec((_BM, d1), lambda i: (i, 0)),
            pl.BlockSpec((2, _BM, d2 // 2), lambda i: (0, i, 0)),
            pl.BlockSpec((_BM, 1), lambda i: (i, 0)),
            pl.BlockSpec((1, d2), lambda i: (0, 0)),
            pl.BlockSpec((d0 + d1 + d2, dtot), lambda i: (0, 0)),
            pl.BlockSpec((1, d0n), lambda i: (0, 0)),
        ],
        out_specs=[
            pl.BlockSpec((_BM, d0n), lambda i: (i, 0)),
            pl.BlockSpec((_BM, dh), lambda i: (i, 0)),
            pl.BlockSpec((_BM, dh), lambda i: (i, 0)),
        ],
        out_shape=[
            jax.ShapeDtypeStruct((N_PAD, d0n), jnp.float32),
            jax.ShapeDtypeStruct((N_PAD, dh), jnp.float32),
            jax.ShapeDtypeStruct((N_PAD, dh), jnp.float32),
        ],
    )(z0, out1, q, dinv, b2, W, b0n)


def _final_out(z0, q, dinv, b1):
    d0 = z0.shape[1]
    d1 = 2 * q.shape[2]

    def body(z0_ref, q_ref, dv_ref, b_ref, h_ref):
        qs = jnp.concatenate([q_ref[0], q_ref[1]], axis=1)
        o1 = qs * dv_ref[...] + b_ref[...]
        h_ref[...] = jnp.concatenate([z0_ref[...], o1], axis=1)

    return pl.pallas_call(
        body,
        grid=(N_PAD // _BM,),
        in_specs=[
            pl.BlockSpec((_BM, d0), lambda i: (i, 0)),
            pl.BlockSpec((2, _BM, d1 // 2), lambda i: (0, i, 0)),
            pl.BlockSpec((_BM, 1), lambda i: (i, 0)),
            pl.BlockSpec((1, d1), lambda i: (0, 0)),
        ],
        out_specs=pl.BlockSpec((_BM, d0 + d1), lambda i: (i, 0)),
        out_shape=jax.ShapeDtypeStruct((N_PAD, d0 + d1), jnp.float32),
    )(z0, q, dinv, b1)


def kernel(x, edge_index, conv1_W, conv1_b, block_W, block_b, conv3_W, conv3_b):
    f32 = jnp.float32

    xp = jnp.pad(x, ((0, N_PAD - N_NODES), (0, 0)))
    npad = E_PAD - N_EDGES
    pad_ids = (jnp.arange(npad, dtype=jnp.int32) % PAD_SPREAD) + N_NODES
    src_flat = jnp.concatenate([edge_index[0], pad_ids])
    dst_flat = jnp.concatenate([edge_index[1], pad_ids])
    srcp = src_flat.reshape(NSUB, GROUPS_P // 8, 8 * BP)
    dstp = dst_flat.reshape(NSUB, GROUPS_P, BP)
    dstd = dst_flat.reshape(NCORES * NSUB, GROUPS_D, BATCH)

    ones16 = jnp.ones((BATCH, 16), f32)
    zeros16 = jnp.zeros((N_PAD, 16), f32)

    degp = _sc_deg()(dstd, ones16, zeros16)
    dinv = _dinv_from_deg(degp)

    prop64 = _sc_prop(64)
    prop32 = _sc_prop(32)
    prop16 = _sc_prop(16)

    def props(aL, aR, b1):
        p = prop64(aL, aR, srcp, dstp)
        out1, gL, gR = _combine_mid(p, dinv, b1, 64)
        q = prop32(gL, gR, srcp, dstp)
        return out1, q

    W1 = jnp.concatenate([conv1_W[0], conv1_W[1], conv1_W[2]], axis=1)
    z0, aL, aR = _dense_in(xp, W1, conv1_b[0][None], dinv, 64)
    out1, q = props(aL, aR, conv1_b[1][None])
    b_prev = conv1_b[2][None]

    for i in range(2):
        Wm = jnp.concatenate([block_W[i, 0], block_W[i, 1], block_W[i, 2]],
                             axis=1)
        z0, aL, aR = _combine_dense(z0, out1, q, dinv, b_prev, Wm,
                                    block_b[i, 0][None], 64)
        out1, q = props(aL, aR, block_b[i, 1][None])
        b_prev = block_b[i, 2][None]

    W3 = jnp.concatenate([conv3_W[0], conv3_W[1]], axis=1)
    z0, aL, aR = _combine_dense(z0, out1, q, dinv, b_prev, W3,
                                conv3_b[0][None], 32)
    q = prop16(aL, aR, srcp, dstp)
    out = _final_out(z0, q, dinv, conv3_b[1][None])
    return out[:N_NODES]
